# baseline (speedup 1.0000x reference)
import functools

import jax
import jax.numpy as jnp
from jax import lax
from jax.experimental import pallas as pl
from jax.experimental.pallas import tpu as pltpu
from jax.experimental.pallas import tpu_sc as plsc

_N = 10000
_K = 8
_NP = 10240
_BQ = 512
_NB = _NP // _BQ
_E = _NP * _K
_PADV = 1e6


def _dot(a, b, dims):
    return lax.dot_general(a, b, (dims, ((), ())),
                           preferred_element_type=jnp.float32)


_NRED = 16
_WR = _NP // _NRED


def _knn_body(xb_ref, post_ref, nbr_ref, *, row_off):
    xb = xb_ref[...]
    post = post_ref[...]

    def slab(k):
        p = post[:, k * _WR:(k + 1) * _WR]
        d = (xb[:, 0:1] - p[0:1, :]) ** 2
        d = d + (xb[:, 1:2] - p[1:2, :]) ** 2
        return d + (xb[:, 2:3] - p[2:3, :]) ** 2

    def const(v):
        return jnp.full((_BQ, _WR), float(v), jnp.float32)

    def leaf_merge(a, b):
        (da, oa), (db, ob) = a, b
        t = da <= db
        return (jnp.where(t, da, db), jnp.where(t, oa, ob),
                jnp.where(t, db, da), jnp.where(t, ob, oa))

    def merge2(a, b):
        a1, ao1, a2, ao2 = a
        b1, bo1, b2, bo2 = b
        t1 = a1 <= b1
        m1 = jnp.where(t1, a1, b1)
        o1 = jnp.where(t1, ao1, bo1)
        ls = jnp.where(t1, b1, a1)
        lo = jnp.where(t1, bo1, ao1)
        t2 = a2 <= b2
        c = jnp.where(t2, a2, b2)
        co = jnp.where(t2, ao2, bo2)
        t3 = ls <= c
        return m1, o1, jnp.where(t3, ls, c), jnp.where(t3, lo, co)

    lv = [leaf_merge((slab(2 * j), const(2 * j * _WR)),
                     (slab(2 * j + 1), const((2 * j + 1) * _WR)))
          for j in range(_NRED // 2)]
    while len(lv) > 1:
        lv = [merge2(lv[j], lv[j + 1]) for j in range(0, len(lv), 2)]
    m1, o1, m2, o2 = lv[0]
    iota = lax.broadcasted_iota(jnp.int32, (_BQ, _WR), 1).astype(jnp.float32)
    dm = jnp.concatenate([m1, m2], axis=1)
    oc = jnp.concatenate([o1 + iota, o2 + iota], axis=1)

    rowf = (row_off + pl.program_id(0) * _BQ
            + lax.broadcasted_iota(jnp.int32, (_BQ, 1), 0)).astype(jnp.float32)
    dm = jnp.where(oc == rowf, jnp.inf, dm)

    picks = []
    for t in range(_K):
        m = jnp.min(dm, axis=1, keepdims=True)
        idxf = jnp.min(jnp.where(dm == m, oc, 3.0e38), axis=1, keepdims=True)
        picks.append(idxf)
        if t < _K - 1:
            dm = jnp.where(oc == idxf, jnp.inf, dm)
    nbr_ref[...] = jnp.concatenate(picks, axis=1).astype(jnp.int32)


_NBH = _NB // 2


def _knn(xpad, post, half):
    full = lambda shape: pl.BlockSpec(shape, lambda i: (0, 0))
    off = half * _NBH
    return pl.pallas_call(
        functools.partial(_knn_body, row_off=off * _BQ),
        grid=(_NBH,),
        in_specs=[pl.BlockSpec((_BQ, 3), lambda i: (i + off, 0)),
                  full((3, _NP))],
        out_specs=pl.BlockSpec((_BQ, _K), lambda i: (i, 0)),
        out_shape=jax.ShapeDtypeStruct((_NP // 2, _K), jnp.int32),
    )(xpad, post)


def _u_body(xb_ref, a1_ref, b1_ref, u_ref):
    u_ref[...] = _dot(xb_ref[...], a1_ref[...], ((1,), (0,))) + b1_ref[...]


def _u_all(xpad, a1, b1r):
    full = lambda shape: pl.BlockSpec(shape, lambda i: (0, 0))
    return pl.pallas_call(
        _u_body,
        grid=(_NB,),
        in_specs=[pl.BlockSpec((_BQ, 3), lambda i: (i, 0)),
                  full((3, 32)), full((1, 32))],
        out_specs=pl.BlockSpec((_BQ, 32), lambda i: (i, 0)),
        out_shape=jax.ShapeDtypeStruct((_NP, 32), jnp.float32),
    )(xpad, a1, b1r)


_SC_CORES = 2
_SC_SUBCORES = 16
_NW = _SC_CORES * _SC_SUBCORES
_CH = 128


_NBUF = 3


@functools.lru_cache(maxsize=None)
def _make_gather(d, erows):
    b_per_w = erows // _NW
    n_ch = b_per_w // _CH
    mesh = plsc.VectorSubcoreMesh(core_axis_name="c", subcore_axis_name="s")
    scratch = ([pltpu.VMEM((_CH,), jnp.int32)] * _NBUF
               + [pltpu.VMEM((_CH, d), jnp.float32)] * _NBUF
               + [pltpu.SemaphoreType.DMA] * (3 * _NBUF))

    @functools.partial(
        pl.kernel, mesh=mesh,
        compiler_params=pltpu.CompilerParams(use_tc_tiling_on_sc=False),
        out_type=jax.ShapeDtypeStruct((erows, d), jnp.float32),
        scratch_types=scratch,
    )
    def gather(table_hbm, idx_hbm, out_hbm, *scr):
        idxs, rows = scr[:_NBUF], scr[_NBUF:2 * _NBUF]
        si = scr[2 * _NBUF:2 * _NBUF + _NBUF]
        sg = scr[3 * _NBUF:3 * _NBUF + _NBUF]
        sw = scr[4 * _NBUF:4 * _NBUF + _NBUF]
        wid = lax.axis_index("s") * _SC_CORES + lax.axis_index("c")
        base = wid * b_per_w
        h_idx = [None] * _NBUF
        h_gat = [None] * _NBUF
        h_wr = [None] * _NBUF
        for b in range(min(_NBUF, n_ch)):
            h_idx[b] = pltpu.async_copy(
                idx_hbm.at[pl.ds(base + b * _CH, _CH)], idxs[b], si[b])
        for c in range(n_ch):
            b = c % _NBUF
            if c >= _NBUF:
                h_wr[b].wait()
            h_idx[b].wait()
            h_gat[b] = pltpu.async_copy(table_hbm.at[idxs[b]], rows[b], sg[b])
            if c >= 1:
                b1 = (c - 1) % _NBUF
                h_gat[b1].wait()
                h_wr[b1] = pltpu.async_copy(
                    rows[b1], out_hbm.at[pl.ds(base + (c - 1) * _CH, _CH)],
                    sw[b1])
                nxt = c - 1 + _NBUF
                if nxt < n_ch:
                    h_idx[b1] = pltpu.async_copy(
                        idx_hbm.at[pl.ds(base + nxt * _CH, _CH)], idxs[b1],
                        si[b1])
        bl = (n_ch - 1) % _NBUF
        h_gat[bl].wait()
        h_wr[bl] = pltpu.async_copy(
            rows[bl], out_hbm.at[pl.ds(base + (n_ch - 1) * _CH, _CH)], sw[bl])
        for b in range(min(_NBUF, n_ch)):
            h_wr[b].wait()

    return gather


def _gather32(table, idx):
    return _make_gather(32, idx.shape[0])(table, idx)


def _gather64(table, idx):
    return _make_gather(64, idx.shape[0])(table, idx)


def _conv1_body(uj_ref, xb_ref, w1b_ref, w1a_ref, b1_ref, w2_ref, b2_ref,
                w3a_ref, w3b_ref, b3_ref, h_ref, g_ref):
    xb = xb_ref[...]
    t2 = _dot(xb, w1b_ref[...], ((1,), (0,)))
    mp = uj_ref[...].reshape(_BQ, _K, 32) - t2[:, None, :]
    mp = jnp.maximum(mp, 0.0).reshape(_BQ * _K, 32)
    m = _dot(mp, w2_ref[...], ((1,), (0,))) + b2_ref[...]
    mk = jnp.max(m.reshape(_BQ, _K, 64), axis=1)
    s = jnp.maximum(_dot(xb, w1a_ref[...], ((1,), (0,))) + b1_ref[...], 0.0)
    s = _dot(s, w2_ref[...], ((1,), (0,))) + b2_ref[...]
    h = jnp.maximum(jnp.maximum(mk, s), 0.0)
    h_ref[...] = h
    g_ref[...] = (_dot(h, w3a_ref[...], ((1,), (0,)))
                  + _dot(xb, w3b_ref[...], ((1,), (0,)))
                  + b3_ref[...])


def _conv1(uj, xpad, half, w1b, w1a, b1r, w2, b2r, w3a, w3b, b3r):
    full = lambda shape: pl.BlockSpec(shape, lambda i: (0, 0))
    off = half * _NBH
    return pl.pallas_call(
        _conv1_body,
        grid=(_NBH,),
        in_specs=[pl.BlockSpec((_BQ * _K, 32), lambda i: (i, 0)),
                  pl.BlockSpec((_BQ, 3), lambda i: (i + off, 0)),
                  full((3, 32)), full((3, 32)),
                  full((1, 32)), full((32, 64)), full((1, 64)),
                  full((64, 64)), full((3, 64)), full((1, 64))],
        out_specs=[pl.BlockSpec((_BQ, 64), lambda i: (i, 0)),
                   pl.BlockSpec((_BQ, 64), lambda i: (i, 0))],
        out_shape=[jax.ShapeDtypeStruct((_NP // 2, 64), jnp.float32),
                   jax.ShapeDtypeStruct((_NP // 2, 64), jnp.float32)],
    )(uj, xpad, w1b, w1a, b1r, w2, b2r, w3a, w3b, b3r)


def _conv2_body(gj_ref, h_ref, xb_ref, w3b_ref, w3a_ref, b3_ref,
                w4_ref, b4_ref, wc_ref, bc_ref, out_ref):
    xw = _dot(xb_ref[...], w3b_ref[...], ((1,), (0,)))
    mp = gj_ref[...].reshape(_BQ, _K, 64) - xw[:, None, :]
    mp = jnp.maximum(mp, 0.0).reshape(_BQ * _K, 64)
    m = _dot(mp, w4_ref[...], ((1,), (0,))) + b4_ref[...]
    mk = jnp.max(m.reshape(_BQ, _K, 128), axis=1)
    hb = h_ref[...]
    s = jnp.maximum(_dot(hb, w3a_ref[...], ((1,), (0,))) + b3_ref[...], 0.0)
    s = _dot(s, w4_ref[...], ((1,), (0,))) + b4_ref[...]
    hh = jnp.maximum(jnp.maximum(mk, s), 0.0)
    logits = _dot(hh, wc_ref[...], ((1,), (0,))) + bc_ref[...]
    mx = jnp.max(logits, axis=1, keepdims=True)
    lse = jnp.log(jnp.sum(jnp.exp(logits - mx), axis=1, keepdims=True)) + mx
    out_ref[...] = logits - lse


def _conv2(gj, h, xpad, half, w3b, w3a, b3r, w4, b4r, wc, bcr):
    full = lambda shape: pl.BlockSpec(shape, lambda i: (0, 0))
    off = half * _NBH
    return pl.pallas_call(
        _conv2_body,
        grid=(_NBH,),
        in_specs=[pl.BlockSpec((_BQ * _K, 64), lambda i: (i, 0)),
                  pl.BlockSpec((_BQ, 64), lambda i: (i, 0)),
                  pl.BlockSpec((_BQ, 3), lambda i: (i + off, 0)),
                  full((3, 64)), full((64, 64)),
                  full((1, 64)), full((64, 128)), full((1, 128)),
                  full((128, 5)), full((1, 5))],
        out_specs=pl.BlockSpec((_BQ, 5), lambda i: (i, 0)),
        out_shape=jax.ShapeDtypeStruct((_NP // 2, 5), jnp.float32),
    )(gj, h, xpad, w3b, w3a, b3r, w4, b4r, wc, bcr)


def kernel(x, batch, W1, b1, W2, b2, W3, b3, W4, b4, Wc, bc):
    del batch
    pad = jnp.full((_NP - _N, 3), _PADV, jnp.float32)
    xpad = jnp.concatenate([x, pad], axis=0)
    post = xpad.T
    a1 = W1[:3] + W1[3:]
    w1a, w1b = W1[:3], W1[3:]
    w3a, w3b = W3[:64], W3[64:]
    b1r, b2r, b3r, b4r, bcr = (b[None, :] for b in (b1, b2, b3, b4, bc))

    u = _u_all(xpad, a1, b1r)
    nbr_a = _knn(xpad, post, 0)
    idx_a = nbr_a.reshape(_E // 2)
    uj_a = _gather32(u, idx_a)
    nbr_b = _knn(xpad, post, 1)
    idx_b = nbr_b.reshape(_E // 2)
    uj_b = _gather32(u, idx_b)
    h_a, g_a = _conv1(uj_a, xpad, 0, w1b, w1a, b1r, W2, b2r, w3a, w3b, b3r)
    h_b, g_b = _conv1(uj_b, xpad, 1, w1b, w1a, b1r, W2, b2r, w3a, w3b, b3r)
    g = jnp.concatenate([g_a, g_b], axis=0)
    gj_a = _gather64(g, idx_a)
    gj_b = _gather64(g, idx_b)
    out_a = _conv2(gj_a, h_a, xpad, 0, w3b, w3a, b3r, W4, b4r, Wc, bcr)
    out_b = _conv2(gj_b, h_b, xpad, 1, w3b, w3a, b3r, W4, b4r, Wc, bcr)
    return jnp.concatenate([out_a, out_b], axis=0)[:_N]


# speedup vs baseline: 1.0438x; 1.0438x over previous
import functools

import jax
import jax.numpy as jnp
from jax import lax
from jax.experimental import pallas as pl
from jax.experimental.pallas import tpu as pltpu
from jax.experimental.pallas import tpu_sc as plsc

_N = 10000
_K = 8
_NP = 10240
_BQ = 512
_NB = _NP // _BQ
_E = _NP * _K
_PADV = 1e6


def _dot(a, b, dims):
    return lax.dot_general(a, b, (dims, ((), ())),
                           preferred_element_type=jnp.float32)


_NRED = 16
_WR = _NP // _NRED


def _knn_body(xb_ref, post_ref, a1_ref, b1_ref, nbr_ref, u_ref):
    xb = xb_ref[...]
    post = post_ref[...]

    def slab(k):
        p = post[:, k * _WR:(k + 1) * _WR]
        d = (xb[:, 0:1] - p[0:1, :]) ** 2
        d = d + (xb[:, 1:2] - p[1:2, :]) ** 2
        return d + (xb[:, 2:3] - p[2:3, :]) ** 2

    def const(v):
        return jnp.full((_BQ, _WR), float(v), jnp.float32)

    def leaf_merge(a, b):
        (da, oa), (db, ob) = a, b
        t = da <= db
        return (jnp.where(t, da, db), jnp.where(t, oa, ob),
                jnp.where(t, db, da), jnp.where(t, ob, oa))

    def merge2(a, b):
        a1, ao1, a2, ao2 = a
        b1, bo1, b2, bo2 = b
        t1 = a1 <= b1
        m1 = jnp.where(t1, a1, b1)
        o1 = jnp.where(t1, ao1, bo1)
        ls = jnp.where(t1, b1, a1)
        lo = jnp.where(t1, bo1, ao1)
        t2 = a2 <= b2
        c = jnp.where(t2, a2, b2)
        co = jnp.where(t2, ao2, bo2)
        t3 = ls <= c
        return m1, o1, jnp.where(t3, ls, c), jnp.where(t3, lo, co)

    lv = [leaf_merge((slab(2 * j), const(2 * j * _WR)),
                     (slab(2 * j + 1), const((2 * j + 1) * _WR)))
          for j in range(_NRED // 2)]
    while len(lv) > 1:
        lv = [merge2(lv[j], lv[j + 1]) for j in range(0, len(lv), 2)]
    m1, o1, m2, o2 = lv[0]
    iota = lax.broadcasted_iota(jnp.int32, (_BQ, _WR), 1).astype(jnp.float32)
    dm = jnp.concatenate([m1, m2], axis=1)
    oc = jnp.concatenate([o1 + iota, o2 + iota], axis=1)

    rowf = (pl.program_id(0) * _BQ
            + lax.broadcasted_iota(jnp.int32, (_BQ, 1), 0)).astype(jnp.float32)
    dm = jnp.where(oc == rowf, jnp.inf, dm)

    picks = []
    for t in range(_K):
        m = jnp.min(dm, axis=1, keepdims=True)
        idxf = jnp.min(jnp.where(dm == m, oc, 3.0e38), axis=1, keepdims=True)
        picks.append(idxf)
        if t < _K - 1:
            dm = jnp.where(oc == idxf, jnp.inf, dm)
    nbr_ref[...] = jnp.concatenate(picks, axis=1).astype(jnp.int32)
    u_ref[...] = _dot(xb, a1_ref[...], ((1,), (0,))) + b1_ref[...]


def _knn(xpad, post, a1, b1r):
    full = lambda shape: pl.BlockSpec(shape, lambda i: (0, 0))
    return pl.pallas_call(
        _knn_body,
        grid=(_NB,),
        in_specs=[pl.BlockSpec((_BQ, 3), lambda i: (i, 0)),
                  full((3, _NP)), full((3, 32)), full((1, 32))],
        out_specs=[pl.BlockSpec((_BQ, _K), lambda i: (i, 0)),
                   pl.BlockSpec((_BQ, 32), lambda i: (i, 0))],
        out_shape=[jax.ShapeDtypeStruct((_NP, _K), jnp.int32),
                   jax.ShapeDtypeStruct((_NP, 32), jnp.float32)],
    )(xpad, post, a1, b1r)


_SC_CORES = 2
_SC_SUBCORES = 16
_NW = _SC_CORES * _SC_SUBCORES
_CH = 256


_NBUF = 3


@functools.lru_cache(maxsize=None)
def _make_gather(d):
    b_per_w = _E // _NW
    n_ch = b_per_w // _CH
    mesh = plsc.VectorSubcoreMesh(core_axis_name="c", subcore_axis_name="s")
    scratch = ([pltpu.VMEM((_CH,), jnp.int32)] * _NBUF
               + [pltpu.VMEM((_CH, d), jnp.float32)] * _NBUF
               + [pltpu.SemaphoreType.DMA] * (3 * _NBUF))

    @functools.partial(
        pl.kernel, mesh=mesh,
        compiler_params=pltpu.CompilerParams(use_tc_tiling_on_sc=False),
        out_type=jax.ShapeDtypeStruct((_E, d), jnp.float32),
        scratch_types=scratch,
    )
    def gather(table_hbm, idx_hbm, out_hbm, *scr):
        idxs, rows = scr[:_NBUF], scr[_NBUF:2 * _NBUF]
        si = scr[2 * _NBUF:2 * _NBUF + _NBUF]
        sg = scr[3 * _NBUF:3 * _NBUF + _NBUF]
        sw = scr[4 * _NBUF:4 * _NBUF + _NBUF]
        wid = lax.axis_index("s") * _SC_CORES + lax.axis_index("c")
        base = wid * b_per_w
        h_idx = [None] * _NBUF
        h_gat = [None] * _NBUF
        h_wr = [None] * _NBUF
        for b in range(min(_NBUF, n_ch)):
            h_idx[b] = pltpu.async_copy(
                idx_hbm.at[pl.ds(base + b * _CH, _CH)], idxs[b], si[b])
        for c in range(n_ch):
            b = c % _NBUF
            if c >= _NBUF:
                h_wr[b].wait()
            h_idx[b].wait()
            h_gat[b] = pltpu.async_copy(table_hbm.at[idxs[b]], rows[b], sg[b])
            if c >= 1:
                b1 = (c - 1) % _NBUF
                h_gat[b1].wait()
                h_wr[b1] = pltpu.async_copy(
                    rows[b1], out_hbm.at[pl.ds(base + (c - 1) * _CH, _CH)],
                    sw[b1])
                nxt = c - 1 + _NBUF
                if nxt < n_ch:
                    h_idx[b1] = pltpu.async_copy(
                        idx_hbm.at[pl.ds(base + nxt * _CH, _CH)], idxs[b1],
                        si[b1])
        bl = (n_ch - 1) % _NBUF
        h_gat[bl].wait()
        h_wr[bl] = pltpu.async_copy(
            rows[bl], out_hbm.at[pl.ds(base + (n_ch - 1) * _CH, _CH)], sw[bl])
        for b in range(min(_NBUF, n_ch)):
            h_wr[b].wait()

    return gather


def _gather32(table, idx):
    return _make_gather(32)(table, idx)


def _gather64(table, idx):
    return _make_gather(64)(table, idx)


def _conv1_body(uj_ref, xb_ref, w1b_ref, w1a_ref, b1_ref, w2_ref, b2_ref,
                w3a_ref, w3b_ref, b3_ref, h_ref, g_ref):
    xb = xb_ref[...]
    t2 = _dot(xb, w1b_ref[...], ((1,), (0,)))
    mp = uj_ref[...].reshape(_BQ, _K, 32) - t2[:, None, :]
    mp = jnp.maximum(mp, 0.0).reshape(_BQ * _K, 32)
    m = _dot(mp, w2_ref[...], ((1,), (0,))) + b2_ref[...]
    mk = jnp.max(m.reshape(_BQ, _K, 64), axis=1)
    s = jnp.maximum(_dot(xb, w1a_ref[...], ((1,), (0,))) + b1_ref[...], 0.0)
    s = _dot(s, w2_ref[...], ((1,), (0,))) + b2_ref[...]
    h = jnp.maximum(jnp.maximum(mk, s), 0.0)
    h_ref[...] = h
    g_ref[...] = (_dot(h, w3a_ref[...], ((1,), (0,)))
                  + _dot(xb, w3b_ref[...], ((1,), (0,)))
                  + b3_ref[...])


def _conv1(uj, xpad, w1b, w1a, b1r, w2, b2r, w3a, w3b, b3r):
    full = lambda shape: pl.BlockSpec(shape, lambda i: (0, 0))
    return pl.pallas_call(
        _conv1_body,
        grid=(_NB,),
        in_specs=[pl.BlockSpec((_BQ * _K, 32), lambda i: (i, 0)),
                  pl.BlockSpec((_BQ, 3), lambda i: (i, 0)),
                  full((3, 32)), full((3, 32)),
                  full((1, 32)), full((32, 64)), full((1, 64)),
                  full((64, 64)), full((3, 64)), full((1, 64))],
        out_specs=[pl.BlockSpec((_BQ, 64), lambda i: (i, 0)),
                   pl.BlockSpec((_BQ, 64), lambda i: (i, 0))],
        out_shape=[jax.ShapeDtypeStruct((_NP, 64), jnp.float32),
                   jax.ShapeDtypeStruct((_NP, 64), jnp.float32)],
    )(uj, xpad, w1b, w1a, b1r, w2, b2r, w3a, w3b, b3r)


def _conv2_body(gj_ref, h_ref, xb_ref, w3b_ref, w3a_ref, b3_ref,
                w4_ref, b4_ref, wc_ref, bc_ref, out_ref):
    xw = _dot(xb_ref[...], w3b_ref[...], ((1,), (0,)))
    mp = gj_ref[...].reshape(_BQ, _K, 64) - xw[:, None, :]
    mp = jnp.maximum(mp, 0.0).reshape(_BQ * _K, 64)
    m = _dot(mp, w4_ref[...], ((1,), (0,))) + b4_ref[...]
    mk = jnp.max(m.reshape(_BQ, _K, 128), axis=1)
    hb = h_ref[...]
    s = jnp.maximum(_dot(hb, w3a_ref[...], ((1,), (0,))) + b3_ref[...], 0.0)
    s = _dot(s, w4_ref[...], ((1,), (0,))) + b4_ref[...]
    hh = jnp.maximum(jnp.maximum(mk, s), 0.0)
    logits = _dot(hh, wc_ref[...], ((1,), (0,))) + bc_ref[...]
    mx = jnp.max(logits, axis=1, keepdims=True)
    lse = jnp.log(jnp.sum(jnp.exp(logits - mx), axis=1, keepdims=True)) + mx
    out_ref[...] = logits - lse


def _conv2(gj, h, xpad, w3b, w3a, b3r, w4, b4r, wc, bcr):
    full = lambda shape: pl.BlockSpec(shape, lambda i: (0, 0))
    return pl.pallas_call(
        _conv2_body,
        grid=(_NB,),
        in_specs=[pl.BlockSpec((_BQ * _K, 64), lambda i: (i, 0)),
                  pl.BlockSpec((_BQ, 64), lambda i: (i, 0)),
                  pl.BlockSpec((_BQ, 3), lambda i: (i, 0)),
                  full((3, 64)), full((64, 64)),
                  full((1, 64)), full((64, 128)), full((1, 128)),
                  full((128, 5)), full((1, 5))],
        out_specs=pl.BlockSpec((_BQ, 5), lambda i: (i, 0)),
        out_shape=jax.ShapeDtypeStruct((_NP, 5), jnp.float32),
    )(gj, h, xpad, w3b, w3a, b3r, w4, b4r, wc, bcr)


def kernel(x, batch, W1, b1, W2, b2, W3, b3, W4, b4, Wc, bc):
    del batch
    pad = jnp.full((_NP - _N, 3), _PADV, jnp.float32)
    xpad = jnp.concatenate([x, pad], axis=0)
    post = xpad.T
    a1 = W1[:3] + W1[3:]
    w1a, w1b = W1[:3], W1[3:]
    w3a, w3b = W3[:64], W3[64:]
    b1r, b2r, b3r, b4r, bcr = (b[None, :] for b in (b1, b2, b3, b4, bc))

    nbr, u = _knn(xpad, post, a1, b1r)
    idx = nbr.reshape(_E)
    uj = _gather32(u, idx)
    h, g = _conv1(uj, xpad, w1b, w1a, b1r, W2, b2r, w3a, w3b, b3r)
    gj = _gather64(g, idx)
    out = _conv2(gj, h, xpad, w3b, w3a, b3r, W4, b4r, Wc, bcr)
    return out[:_N]
